# Initial kernel scaffold; baseline (speedup 1.0000x reference)
import jax
import jax.numpy as jnp
from jax.experimental import pallas as pl


def kernel(x, edge_index, W1, a_src1, a_dst1, b1, W2, a_src2, a_dst2, b2):
    raise NotImplementedError("write your pallas kernel here")


# speedup vs baseline: 1.0001x; 1.0001x over previous
import jax
import jax.numpy as jnp
from jax.experimental import pallas as pl

N_NODES = 10000
HEADS = 8
HID = 64
OUT_DIM = 128


def _lsm_body(x_ref, o_ref):
    x = x_ref[...]
    m = jnp.max(x, axis=1, keepdims=True)
    s = x - m
    lse = jnp.log(jnp.sum(jnp.exp(s), axis=1, keepdims=True))
    o_ref[...] = s - lse


def _gat_layer(h, src, dst, W, a_s, a_d, b, heads, out_dim, concat):
    N = h.shape[0]
    hW = (h @ W).reshape(N, heads, out_dim)
    alpha_s = (hW * a_s[None, :, :]).sum(axis=-1)
    alpha_d = (hW * a_d[None, :, :]).sum(axis=-1)
    e = alpha_s[src] + alpha_d[dst]
    e = jax.nn.leaky_relu(e, negative_slope=0.2)
    e_max = jax.ops.segment_max(e, dst, num_segments=N)
    e_max = jnp.where(jnp.isfinite(e_max), e_max, 0.0)
    ex = jnp.exp(e - e_max[dst])
    denom = jax.ops.segment_sum(ex, dst, num_segments=N)
    alpha = ex / (denom[dst] + 1e-16)
    msg = hW[src] * alpha[:, :, None]
    out = jax.ops.segment_sum(msg, dst, num_segments=N)
    if concat:
        out = out.reshape(N, heads * out_dim)
    else:
        out = out.mean(axis=1)
    return out + b


def kernel(x, edge_index, W1, a_src1, a_dst1, b1, W2, a_src2, a_dst2, b2):
    src = edge_index[0]
    dst = edge_index[1]
    h = _gat_layer(x, src, dst, W1, a_src1, a_dst1, b1, HEADS, HID, True)
    h = jax.nn.elu(h)
    h = _gat_layer(h, src, dst, W2, a_src2, a_dst2, b2, 1, OUT_DIM, False)
    return pl.pallas_call(
        _lsm_body,
        grid=(10,),
        in_specs=[pl.BlockSpec((1000, OUT_DIM), lambda i: (i, 0))],
        out_specs=pl.BlockSpec((1000, OUT_DIM), lambda i: (i, 0)),
        out_shape=jax.ShapeDtypeStruct((N_NODES, OUT_DIM), jnp.float32),
    )(h)


# speedup vs baseline: 15.4833x; 15.4814x over previous
import jax
import jax.numpy as jnp
from jax import lax
from jax.experimental import pallas as pl
from jax.experimental.pallas import tpu as pltpu
from jax.experimental.pallas import tpu_sc as plsc

N = 10000
E = 320000
IN_D = 128
HID_D = 64
NHEAD = 8
OUT_D = 128

NC = 2
NS = 16
NW = NC * NS
EPT = E // NW
BLK = 80
NBLK = EPT // BLK
CHK = 25
NCHK = NBLK // CHK
NP = 10240
NROW = NP // NS
ZR = 64

_f32 = jnp.float32


def _tc1_body(x_ref, w1_ref, asr_ref, adr_ref, *refs):
    hw_refs = refs[0:NHEAD]
    as_refs = refs[NHEAD:2 * NHEAD]
    ad_refs = refs[2 * NHEAD:3 * NHEAD]
    cc_ref = refs[3 * NHEAD]
    mxs_ref, mxd_ref = refs[3 * NHEAD + 1], refs[3 * NHEAD + 2]
    i = pl.program_id(0)
    nb = pl.num_programs(0)
    hb = jnp.dot(x_ref[...], w1_ref[...], preferred_element_type=_f32)
    for h in range(NHEAD):
        hh = hb[:, h * HID_D:(h + 1) * HID_D]
        hw_refs[h][...] = hh
        sv = jnp.sum(hh * asr_ref[h:h + 1, :], axis=1, keepdims=True)
        dv = jnp.sum(hh * adr_ref[h:h + 1, :], axis=1, keepdims=True)
        as_refs[h][...] = sv
        ad_refs[h][...] = dv
        ms = jnp.max(sv)
        md = jnp.max(dv)

        @pl.when(i == 0)
        def _(h=h, ms=ms, md=md):
            mxs_ref[h:h + 1, :] = jnp.full((1, 128), ms, _f32)
            mxd_ref[h:h + 1, :] = jnp.full((1, 128), md, _f32)

        @pl.when(i > 0)
        def _(h=h, ms=ms, md=md):
            mxs_ref[h:h + 1, :] = jnp.maximum(mxs_ref[h:h + 1, :], ms)
            mxd_ref[h:h + 1, :] = jnp.maximum(mxd_ref[h:h + 1, :], md)

    @pl.when(i == nb - 1)
    def _():
        t = mxs_ref[...] + mxd_ref[...]
        cc_ref[...] = jnp.maximum(t, 0.2 * t)


def _tc1(x, W1, a_src1, a_dst1):
    bn = 1000
    grid = (N // bn,)
    outs = (
        [jax.ShapeDtypeStruct((N, HID_D), _f32)] * NHEAD
        + [jax.ShapeDtypeStruct((N, 1), _f32)] * (2 * NHEAD)
        + [jax.ShapeDtypeStruct((NHEAD, 128), _f32)]
    )
    out_specs = (
        [pl.BlockSpec((bn, HID_D), lambda i: (i, 0))] * NHEAD
        + [pl.BlockSpec((bn, 1), lambda i: (i, 0))] * (2 * NHEAD)
        + [pl.BlockSpec((NHEAD, 128), lambda i: (0, 0))]
    )
    return pl.pallas_call(
        _tc1_body,
        grid=grid,
        in_specs=[
            pl.BlockSpec((bn, IN_D), lambda i: (i, 0)),
            pl.BlockSpec((IN_D, NHEAD * HID_D), lambda i: (0, 0)),
            pl.BlockSpec((NHEAD, HID_D), lambda i: (0, 0)),
            pl.BlockSpec((NHEAD, HID_D), lambda i: (0, 0)),
        ],
        out_specs=out_specs,
        out_shape=outs,
        scratch_shapes=[
            pltpu.VMEM((NHEAD, 128), _f32),
            pltpu.VMEM((NHEAD, 128), _f32),
        ],
    )(x, W1, a_src1, a_dst1)


NDR = NP // 16
DRW = NROW // 16


def _make_sc_agg(H, D):
    GD = D // 16
    mesh = plsc.VectorSubcoreMesh(core_axis_name="c", subcore_axis_name="s")

    def body(src_h, dst_h, cc_h, *rest):
        as_hs = rest[0:H]
        ad_hs = rest[H:2 * H]
        hw_hs = rest[2 * H:3 * H]
        acc_o = rest[3 * H]
        den_o = rest[3 * H + 1]
        (srcb, dstb, as_v, ad_v, cc_v, rows, exb, denb, rows_den, zbuf,
         zden, acc_sh, den_sh, sem) = rest[3 * H + 2:]
        c = lax.axis_index("c")
        s = lax.axis_index("s")
        wid = s * NC + c
        pltpu.sync_copy(cc_h, cc_v)
        lanes = lax.iota(jnp.int32, 16)

        def zrow(r, carry):
            for k in range(GD):
                zbuf[r, pl.ds(k * 16, 16)] = jnp.zeros((16,), _f32)
            return carry

        lax.fori_loop(0, ZR, zrow, 0)

        def zdrow(r, carry):
            zden[r, pl.ds(0, 16)] = jnp.zeros((16,), _f32)
            return carry

        lax.fori_loop(0, DRW, zdrow, 0)

        for h in range(H):
            pltpu.sync_copy(as_hs[h], as_v)
            pltpu.sync_copy(ad_hs[h], ad_v)
            ccx = cc_v[pl.ds(h * 16, 16)]
            for z in range(NROW // ZR):
                pltpu.sync_copy(zbuf, acc_sh.at[pl.ds(s * NROW + z * ZR, ZR)])
            pltpu.sync_copy(zden, den_sh.at[pl.ds(s * DRW, DRW)])
            plsc.subcore_barrier()
            hw = hw_hs[h]

            def blk(b, carry, ccx=ccx, hw=hw):
                for g in range(BLK // 16):
                    sidx = srcb[b, pl.ds(g * 16, 16)]
                    didx = dstb[b, pl.ds(g * 16, 16)]
                    sv = plsc.load_gather(as_v, [sidx])
                    dv = plsc.load_gather(ad_v, [didx])
                    t = sv + dv
                    e = jnp.maximum(t, 0.2 * t)
                    exb[pl.ds(g * 16, 16)] = jnp.exp(e - ccx)
                    denb[pl.ds(g * 16, 16)] = lax.shift_right_logical(didx, 4)

                def zr2(i, inner):
                    rows_den[i, pl.ds(0, 16)] = jnp.zeros((16,), _f32)
                    return inner

                lax.fori_loop(0, BLK, zr2, 0)
                for g in range(BLK // 16):
                    didx = dstb[b, pl.ds(g * 16, 16)]
                    ex = exb[pl.ds(g * 16, 16)]
                    plsc.store_scatter(
                        rows_den,
                        [g * 16 + lanes, jnp.bitwise_and(didx, 15)], ex)
                pltpu.async_copy(hw.at[srcb.at[b]], rows, sem).wait()

                def srow(i, inner):
                    bc = plsc.load_gather(exb, [jnp.full((16,), i, jnp.int32)])
                    for k in range(GD):
                        rows[i, pl.ds(k * 16, 16)] = (
                            rows[i, pl.ds(k * 16, 16)] * bc)
                    return inner

                lax.fori_loop(0, BLK, srow, 0)
                pltpu.sync_copy(rows, acc_sh.at[dstb.at[b]], add=True)
                pltpu.sync_copy(rows_den, den_sh.at[denb], add=True)
                return carry

            for kc in range(NCHK):
                pltpu.sync_copy(src_h.at[wid, pl.ds(kc * CHK, CHK)], srcb)
                pltpu.sync_copy(dst_h.at[wid, pl.ds(kc * CHK, CHK)], dstb)
                lax.fori_loop(0, CHK, blk, 0)
            plsc.subcore_barrier()
            off = (c * H + h) * NP + s * NROW
            pltpu.sync_copy(acc_sh.at[pl.ds(s * NROW, NROW)],
                            acc_o.at[pl.ds(off, NROW)])
            doff = (c * H + h) * NDR + s * DRW
            pltpu.sync_copy(den_sh.at[pl.ds(s * DRW, DRW)],
                            den_o.at[pl.ds(doff, DRW)])

    scratch = [
        pltpu.VMEM((CHK, BLK), jnp.int32),
        pltpu.VMEM((CHK, BLK), jnp.int32),
        pltpu.VMEM((NP,), _f32),
        pltpu.VMEM((NP,), _f32),
        pltpu.VMEM((H * 16,), _f32),
        pltpu.VMEM((BLK, D), _f32),
        pltpu.VMEM((128,), _f32),
        pltpu.VMEM((BLK,), jnp.int32),
        pltpu.VMEM((BLK, 16), _f32),
        pltpu.VMEM((ZR, D), _f32),
        pltpu.VMEM((DRW, 16), _f32),
        pltpu.VMEM_SHARED((NP, D), _f32),
        pltpu.VMEM_SHARED((NDR, 16), _f32),
        pltpu.SemaphoreType.DMA,
    ]
    return pl.kernel(
        body,
        out_type=(jax.ShapeDtypeStruct((2 * H * NP, D), _f32),
                  jax.ShapeDtypeStruct((2 * H * NDR, 16), _f32)),
        mesh=mesh,
        scratch_types=scratch,
        compiler_params=pltpu.CompilerParams(
            needs_layout_passes=False, use_tc_tiling_on_sc=False),
    )


def _tc2_body(acc_ref, den_ref, b1_ref, w2_ref, asw_ref, adw_ref,
              hw2_ref, as2_ref, ad2_ref, cc2_ref, mxs_ref, mxd_ref):
    i = pl.program_id(0)
    nb = pl.num_programs(0)
    cols = []
    for h in range(NHEAD):
        num = acc_ref[0, h, :, :] + acc_ref[1, h, :, :]
        den = den_ref[0, h, :, :] + den_ref[1, h, :, :]
        o = num / (den + 1e-16) + b1_ref[0:1, h * HID_D:(h + 1) * HID_D]
        o = jnp.where(o > 0, o, jnp.exp(o) - 1.0)
        cols.append(o)
    h1 = jnp.concatenate(cols, axis=1)
    hw2 = jnp.dot(h1, w2_ref[...], preferred_element_type=_f32)
    hw2_ref[...] = hw2
    sv = jnp.sum(hw2 * asw_ref[...], axis=1, keepdims=True)
    dv = jnp.sum(hw2 * adw_ref[...], axis=1, keepdims=True)
    as2_ref[...] = sv
    ad2_ref[...] = dv
    ms = jnp.max(sv)
    md = jnp.max(dv)

    @pl.when(i == 0)
    def _():
        mxs_ref[...] = jnp.full((8, 128), ms, _f32)
        mxd_ref[...] = jnp.full((8, 128), md, _f32)

    @pl.when(i > 0)
    def _():
        mxs_ref[...] = jnp.maximum(mxs_ref[...], ms)
        mxd_ref[...] = jnp.maximum(mxd_ref[...], md)

    @pl.when(i == nb - 1)
    def _():
        t = mxs_ref[...] + mxd_ref[...]
        cc2_ref[...] = jnp.maximum(t, 0.2 * t)


def _tc2(acc1, den1, b1, W2, a_src2, a_dst2):
    bn = 1000
    grid = (N // bn,)
    return pl.pallas_call(
        _tc2_body,
        grid=grid,
        in_specs=[
            pl.BlockSpec((2, NHEAD, bn, HID_D), lambda i: (0, 0, i, 0)),
            pl.BlockSpec((2, NHEAD, bn, 1), lambda i: (0, 0, i, 0)),
            pl.BlockSpec((1, NHEAD * HID_D), lambda i: (0, 0)),
            pl.BlockSpec((NHEAD * HID_D, OUT_D), lambda i: (0, 0)),
            pl.BlockSpec((1, OUT_D), lambda i: (0, 0)),
            pl.BlockSpec((1, OUT_D), lambda i: (0, 0)),
        ],
        out_specs=[
            pl.BlockSpec((bn, OUT_D), lambda i: (i, 0)),
            pl.BlockSpec((bn, 1), lambda i: (i, 0)),
            pl.BlockSpec((bn, 1), lambda i: (i, 0)),
            pl.BlockSpec((8, 128), lambda i: (0, 0)),
        ],
        out_shape=[
            jax.ShapeDtypeStruct((N, OUT_D), _f32),
            jax.ShapeDtypeStruct((N, 1), _f32),
            jax.ShapeDtypeStruct((N, 1), _f32),
            jax.ShapeDtypeStruct((8, 128), _f32),
        ],
        scratch_shapes=[
            pltpu.VMEM((8, 128), _f32),
            pltpu.VMEM((8, 128), _f32),
        ],
    )(acc1, den1, b1, W2, a_src2, a_dst2)


def _tc3_body(acc_ref, den_ref, b2_ref, o_ref):
    num = acc_ref[0, :, :] + acc_ref[1, :, :]
    den = den_ref[0, :, :] + den_ref[1, :, :]
    o = num / (den + 1e-16) + b2_ref[0:1, :]
    m = jnp.max(o, axis=1, keepdims=True)
    sh = o - m
    lse = jnp.log(jnp.sum(jnp.exp(sh), axis=1, keepdims=True))
    o_ref[...] = sh - lse


def _tc3(acc2, den2, b2):
    bn = 1000
    grid = (N // bn,)
    return pl.pallas_call(
        _tc3_body,
        grid=grid,
        in_specs=[
            pl.BlockSpec((2, bn, OUT_D), lambda i: (0, i, 0)),
            pl.BlockSpec((2, bn, 1), lambda i: (0, i, 0)),
            pl.BlockSpec((1, OUT_D), lambda i: (0, 0)),
        ],
        out_specs=pl.BlockSpec((bn, OUT_D), lambda i: (i, 0)),
        out_shape=jax.ShapeDtypeStruct((N, OUT_D), _f32),
    )(acc2, den2, b2)


def kernel(x, edge_index, W1, a_src1, a_dst1, b1, W2, a_src2, a_dst2, b2):
    src = edge_index[0].astype(jnp.int32).reshape(NW, NBLK, BLK)
    dst = edge_index[1].astype(jnp.int32).reshape(NW, NBLK, BLK)

    t1 = _tc1(x, W1, a_src1, a_dst1)
    hw1 = t1[0:NHEAD]
    as1 = [jnp.pad(a.reshape(N), (0, NP - N)) for a in t1[NHEAD:2 * NHEAD]]
    ad1 = [jnp.pad(a.reshape(N), (0, NP - N)) for a in t1[2 * NHEAD:3 * NHEAD]]
    cc1x = t1[3 * NHEAD][:, 0:16].reshape(NHEAD * 16)

    agg1 = _make_sc_agg(NHEAD, HID_D)
    acc1, den1 = agg1(src, dst, cc1x, *as1, *ad1, *hw1)
    acc1 = acc1.reshape(2, NHEAD, NP, HID_D)
    den1 = den1.reshape(2, NHEAD, NP, 1)

    hw2, as2, ad2, cc2f = _tc2(acc1, den1, b1.reshape(1, NHEAD * HID_D), W2,
                               a_src2, a_dst2)
    cc2x = cc2f[0, 0:16]

    agg2 = _make_sc_agg(1, OUT_D)
    acc2, den2 = agg2(src, dst, cc2x, jnp.pad(as2.reshape(N), (0, NP - N)),
                      jnp.pad(ad2.reshape(N), (0, NP - N)), hw2)
    acc2 = acc2.reshape(2, NP, OUT_D)
    den2 = den2.reshape(2, NP, 1)

    return _tc3(acc2, den2, b2.reshape(1, OUT_D))


# speedup vs baseline: 19.8501x; 1.2820x over previous
import jax
import jax.numpy as jnp
from jax import lax
from jax.experimental import pallas as pl
from jax.experimental.pallas import tpu as pltpu
from jax.experimental.pallas import tpu_sc as plsc

N = 10000
E = 320000
IN_D = 128
HID_D = 64
NHEAD = 8
OUT_D = 128

NC = 2
NS = 16
NW = NC * NS
EPT = E // NW
BLK = 80
NBLK = EPT // BLK
CHK = 25
NCHK = NBLK // CHK
NP = 10240
NROW = NP // NS
ZR = 64

_f32 = jnp.float32


def _tc1_body(x_ref, w1_ref, asr_ref, adr_ref, *refs):
    hw_refs = refs[0:NHEAD]
    as_refs = refs[NHEAD:2 * NHEAD]
    ad_refs = refs[2 * NHEAD:3 * NHEAD]
    cc_ref = refs[3 * NHEAD]
    mxs_ref, mxd_ref = refs[3 * NHEAD + 1], refs[3 * NHEAD + 2]
    i = pl.program_id(0)
    nb = pl.num_programs(0)
    hb = jnp.dot(x_ref[...], w1_ref[...], preferred_element_type=_f32)
    for h in range(NHEAD):
        hh = hb[:, h * HID_D:(h + 1) * HID_D]
        hw_refs[h][...] = hh
        sv = jnp.sum(hh * asr_ref[h:h + 1, :], axis=1, keepdims=True)
        dv = jnp.sum(hh * adr_ref[h:h + 1, :], axis=1, keepdims=True)
        as_refs[h][...] = sv
        ad_refs[h][...] = dv
        ms = jnp.max(sv)
        md = jnp.max(dv)

        @pl.when(i == 0)
        def _(h=h, ms=ms, md=md):
            mxs_ref[h:h + 1, :] = jnp.full((1, 128), ms, _f32)
            mxd_ref[h:h + 1, :] = jnp.full((1, 128), md, _f32)

        @pl.when(i > 0)
        def _(h=h, ms=ms, md=md):
            mxs_ref[h:h + 1, :] = jnp.maximum(mxs_ref[h:h + 1, :], ms)
            mxd_ref[h:h + 1, :] = jnp.maximum(mxd_ref[h:h + 1, :], md)

    @pl.when(i == nb - 1)
    def _():
        t = mxs_ref[...] + mxd_ref[...]
        cc_ref[...] = jnp.maximum(t, 0.2 * t)


def _tc1(x, W1, a_src1, a_dst1):
    bn = 1000
    grid = (N // bn,)
    outs = (
        [jax.ShapeDtypeStruct((N, HID_D), _f32)] * NHEAD
        + [jax.ShapeDtypeStruct((N, 1), _f32)] * (2 * NHEAD)
        + [jax.ShapeDtypeStruct((NHEAD, 128), _f32)]
    )
    out_specs = (
        [pl.BlockSpec((bn, HID_D), lambda i: (i, 0))] * NHEAD
        + [pl.BlockSpec((bn, 1), lambda i: (i, 0))] * (2 * NHEAD)
        + [pl.BlockSpec((NHEAD, 128), lambda i: (0, 0))]
    )
    return pl.pallas_call(
        _tc1_body,
        grid=grid,
        in_specs=[
            pl.BlockSpec((bn, IN_D), lambda i: (i, 0)),
            pl.BlockSpec((IN_D, NHEAD * HID_D), lambda i: (0, 0)),
            pl.BlockSpec((NHEAD, HID_D), lambda i: (0, 0)),
            pl.BlockSpec((NHEAD, HID_D), lambda i: (0, 0)),
        ],
        out_specs=out_specs,
        out_shape=outs,
        scratch_shapes=[
            pltpu.VMEM((NHEAD, 128), _f32),
            pltpu.VMEM((NHEAD, 128), _f32),
        ],
    )(x, W1, a_src1, a_dst1)


NDR = NP // 16
DRW = NROW // 16


def _make_sc_agg(H, D):
    GD = D // 16
    mesh = plsc.VectorSubcoreMesh(core_axis_name="c", subcore_axis_name="s")

    def body(src_h, dst_h, cc_h, *rest):
        as_hs = rest[0:H]
        ad_hs = rest[H:2 * H]
        hw_hs = rest[2 * H:3 * H]
        acc_o = rest[3 * H]
        den_o = rest[3 * H + 1]
        (srcb, dstb, as_v, ad_v, cc_v, rows, exb, denb, rows_den, zbuf,
         zden, acc_sh, den_sh, sem_g) = rest[3 * H + 2:]
        c = lax.axis_index("c")
        s = lax.axis_index("s")
        wid = s * NC + c
        pltpu.sync_copy(cc_h, cc_v)
        lanes = lax.iota(jnp.int32, 16)

        def zrow(r, carry):
            for k in range(GD):
                zbuf[r, pl.ds(k * 16, 16)] = jnp.zeros((16,), _f32)
            return carry

        lax.fori_loop(0, ZR, zrow, 0)

        def zdrow(r, carry):
            zden[r, pl.ds(0, 16)] = jnp.zeros((16,), _f32)
            return carry

        lax.fori_loop(0, DRW, zdrow, 0)

        for h in range(H):
            pltpu.sync_copy(as_hs[h], as_v)
            pltpu.sync_copy(ad_hs[h], ad_v)
            ccx = cc_v[pl.ds(h * 16, 16)]
            for z in range(NROW // ZR):
                pltpu.sync_copy(zbuf, acc_sh.at[pl.ds(s * NROW + z * ZR, ZR)])
            pltpu.sync_copy(zden, den_sh.at[pl.ds(s * DRW, DRW)])
            plsc.subcore_barrier()
            hw = hw_hs[h]

            def blk(b, carry, ccx=ccx, hw=hw):
                p = jnp.bitwise_and(b, 1)
                q = 1 - p
                for g in range(BLK // 16):
                    sidx = srcb[b, pl.ds(g * 16, 16)]
                    didx = dstb[b, pl.ds(g * 16, 16)]
                    sv = plsc.load_gather(as_v, [sidx])
                    dv = plsc.load_gather(ad_v, [didx])
                    t = sv + dv
                    e = jnp.maximum(t, 0.2 * t)
                    exb[pl.ds(g * 16, 16)] = jnp.exp(e - ccx)
                    denb[pl.ds(g * 16, 16)] = lax.shift_right_logical(didx, 4)
                pltpu.make_async_copy(hw.at[srcb.at[b]], rows.at[p],
                                      sem_g.at[p]).wait()

                @pl.when(b < CHK - 1)
                def _():
                    pltpu.async_copy(hw.at[srcb.at[b + 1]], rows.at[q],
                                     sem_g.at[q])

                def zr2(j, inner):
                    rows_den[j, pl.ds(0, 16)] = jnp.zeros((16,), _f32)
                    return inner

                lax.fori_loop(0, BLK, zr2, 0)
                for g in range(BLK // 16):
                    didx = dstb[b, pl.ds(g * 16, 16)]
                    ex = exb[pl.ds(g * 16, 16)]
                    plsc.store_scatter(
                        rows_den,
                        [g * 16 + lanes, jnp.bitwise_and(didx, 15)], ex)
                rp = rows.at[p]

                def srow(i, inner):
                    bc = plsc.load_gather(exb, [jnp.full((16,), i, jnp.int32)])
                    for k in range(GD):
                        rp[i, pl.ds(k * 16, 16)] = (
                            rp[i, pl.ds(k * 16, 16)] * bc)
                    return inner

                lax.fori_loop(0, BLK, srow, 0)
                pltpu.sync_copy(rp, acc_sh.at[dstb.at[b]], add=True)
                pltpu.sync_copy(rows_den, den_sh.at[denb], add=True)
                return carry

            def chunk(kc, carry, hw=hw, blk=blk):
                pltpu.sync_copy(src_h.at[wid, pl.ds(kc * CHK, CHK)], srcb)
                pltpu.sync_copy(dst_h.at[wid, pl.ds(kc * CHK, CHK)], dstb)
                pltpu.async_copy(hw.at[srcb.at[0]], rows.at[0], sem_g.at[0])
                lax.fori_loop(0, CHK, blk, 0)
                return carry

            lax.fori_loop(0, NCHK, chunk, 0)
            plsc.subcore_barrier()
            off = (c * H + h) * NP + s * NROW
            pltpu.sync_copy(acc_sh.at[pl.ds(s * NROW, NROW)],
                            acc_o.at[pl.ds(off, NROW)])
            doff = (c * H + h) * NDR + s * DRW
            pltpu.sync_copy(den_sh.at[pl.ds(s * DRW, DRW)],
                            den_o.at[pl.ds(doff, DRW)])

    scratch = [
        pltpu.VMEM((CHK, BLK), jnp.int32),
        pltpu.VMEM((CHK, BLK), jnp.int32),
        pltpu.VMEM((NP,), _f32),
        pltpu.VMEM((NP,), _f32),
        pltpu.VMEM((H * 16,), _f32),
        pltpu.VMEM((2, BLK, D), _f32),
        pltpu.VMEM((128,), _f32),
        pltpu.VMEM((BLK,), jnp.int32),
        pltpu.VMEM((BLK, 16), _f32),
        pltpu.VMEM((ZR, D), _f32),
        pltpu.VMEM((DRW, 16), _f32),
        pltpu.VMEM_SHARED((NP, D), _f32),
        pltpu.VMEM_SHARED((NDR, 16), _f32),
        pltpu.SemaphoreType.DMA((2,)),
    ]
    return pl.kernel(
        body,
        out_type=(jax.ShapeDtypeStruct((2 * H * NP, D), _f32),
                  jax.ShapeDtypeStruct((2 * H * NDR, 16), _f32)),
        mesh=mesh,
        scratch_types=scratch,
        compiler_params=pltpu.CompilerParams(
            needs_layout_passes=False, use_tc_tiling_on_sc=False),
    )


def _tc2_body(acc_ref, den_ref, b1_ref, w2_ref, asw_ref, adw_ref,
              hw2a_ref, hw2b_ref, as2_ref, ad2_ref, cc2_ref,
              mxs_ref, mxd_ref):
    i = pl.program_id(0)
    nb = pl.num_programs(0)
    cols = []
    for h in range(NHEAD):
        num = acc_ref[0, h, :, :] + acc_ref[1, h, :, :]
        den = den_ref[0, h, :, :] + den_ref[1, h, :, :]
        o = num / (den + 1e-16) + b1_ref[0:1, h * HID_D:(h + 1) * HID_D]
        o = jnp.where(o > 0, o, jnp.exp(o) - 1.0)
        cols.append(o)
    h1 = jnp.concatenate(cols, axis=1)
    hw2 = jnp.dot(h1, w2_ref[...], preferred_element_type=_f32)
    hw2a_ref[...] = hw2[:, 0:OUT_D // 2]
    hw2b_ref[...] = hw2[:, OUT_D // 2:OUT_D]
    sv = jnp.sum(hw2 * asw_ref[...], axis=1, keepdims=True)
    dv = jnp.sum(hw2 * adw_ref[...], axis=1, keepdims=True)
    as2_ref[...] = sv
    ad2_ref[...] = dv
    ms = jnp.max(sv)
    md = jnp.max(dv)

    @pl.when(i == 0)
    def _():
        mxs_ref[...] = jnp.full((8, 128), ms, _f32)
        mxd_ref[...] = jnp.full((8, 128), md, _f32)

    @pl.when(i > 0)
    def _():
        mxs_ref[...] = jnp.maximum(mxs_ref[...], ms)
        mxd_ref[...] = jnp.maximum(mxd_ref[...], md)

    @pl.when(i == nb - 1)
    def _():
        t = mxs_ref[...] + mxd_ref[...]
        cc2_ref[...] = jnp.maximum(t, 0.2 * t)


def _tc2(acc1, den1, b1, W2, a_src2, a_dst2):
    bn = 1000
    grid = (N // bn,)
    return pl.pallas_call(
        _tc2_body,
        grid=grid,
        in_specs=[
            pl.BlockSpec((2, NHEAD, bn, HID_D), lambda i: (0, 0, i, 0)),
            pl.BlockSpec((2, NHEAD, bn, 1), lambda i: (0, 0, i, 0)),
            pl.BlockSpec((1, NHEAD * HID_D), lambda i: (0, 0)),
            pl.BlockSpec((NHEAD * HID_D, OUT_D), lambda i: (0, 0)),
            pl.BlockSpec((1, OUT_D), lambda i: (0, 0)),
            pl.BlockSpec((1, OUT_D), lambda i: (0, 0)),
        ],
        out_specs=[
            pl.BlockSpec((bn, OUT_D // 2), lambda i: (i, 0)),
            pl.BlockSpec((bn, OUT_D // 2), lambda i: (i, 0)),
            pl.BlockSpec((bn, 1), lambda i: (i, 0)),
            pl.BlockSpec((bn, 1), lambda i: (i, 0)),
            pl.BlockSpec((8, 128), lambda i: (0, 0)),
        ],
        out_shape=[
            jax.ShapeDtypeStruct((N, OUT_D // 2), _f32),
            jax.ShapeDtypeStruct((N, OUT_D // 2), _f32),
            jax.ShapeDtypeStruct((N, 1), _f32),
            jax.ShapeDtypeStruct((N, 1), _f32),
            jax.ShapeDtypeStruct((8, 128), _f32),
        ],
        scratch_shapes=[
            pltpu.VMEM((8, 128), _f32),
            pltpu.VMEM((8, 128), _f32),
        ],
    )(acc1, den1, b1, W2, a_src2, a_dst2)


def _tc3_body(acc_ref, den_ref, b2_ref, o_ref):
    num = jnp.concatenate(
        [acc_ref[0, 0, :, :] + acc_ref[1, 0, :, :],
         acc_ref[0, 1, :, :] + acc_ref[1, 1, :, :]], axis=1)
    den = den_ref[0, :, :] + den_ref[1, :, :]
    o = num / (den + 1e-16) + b2_ref[0:1, :]
    m = jnp.max(o, axis=1, keepdims=True)
    sh = o - m
    lse = jnp.log(jnp.sum(jnp.exp(sh), axis=1, keepdims=True))
    o_ref[...] = sh - lse


def _tc3(acc2, den2, b2):
    bn = 1000
    grid = (N // bn,)
    return pl.pallas_call(
        _tc3_body,
        grid=grid,
        in_specs=[
            pl.BlockSpec((2, 2, bn, OUT_D // 2), lambda i: (0, 0, i, 0)),
            pl.BlockSpec((2, bn, 1), lambda i: (0, i, 0)),
            pl.BlockSpec((1, OUT_D), lambda i: (0, 0)),
        ],
        out_specs=pl.BlockSpec((bn, OUT_D), lambda i: (i, 0)),
        out_shape=jax.ShapeDtypeStruct((N, OUT_D), _f32),
    )(acc2, den2, b2)


def kernel(x, edge_index, W1, a_src1, a_dst1, b1, W2, a_src2, a_dst2, b2):
    src = edge_index[0].astype(jnp.int32).reshape(NW, NBLK, BLK)
    dst = edge_index[1].astype(jnp.int32).reshape(NW, NBLK, BLK)

    t1 = _tc1(x, W1, a_src1, a_dst1)
    hw1 = t1[0:NHEAD]
    as1 = [jnp.pad(a.reshape(N), (0, NP - N)) for a in t1[NHEAD:2 * NHEAD]]
    ad1 = [jnp.pad(a.reshape(N), (0, NP - N)) for a in t1[2 * NHEAD:3 * NHEAD]]
    cc1x = t1[3 * NHEAD][:, 0:16].reshape(NHEAD * 16)

    agg1 = _make_sc_agg(NHEAD, HID_D)
    acc1, den1 = agg1(src, dst, cc1x, *as1, *ad1, *hw1)
    acc1 = acc1.reshape(2, NHEAD, NP, HID_D)
    den1 = den1.reshape(2, NHEAD, NP, 1)

    hw2a, hw2b, as2, ad2, cc2f = _tc2(acc1, den1,
                                      b1.reshape(1, NHEAD * HID_D), W2,
                                      a_src2, a_dst2)
    cc2 = cc2f[0, 0:16]
    cc2x = jnp.concatenate([cc2, cc2])

    as2p = jnp.pad(as2.reshape(N), (0, NP - N))
    ad2p = jnp.pad(ad2.reshape(N), (0, NP - N))
    agg2 = _make_sc_agg(2, OUT_D // 2)
    acc2, den2 = agg2(src, dst, cc2x, as2p, as2p, ad2p, ad2p, hw2a, hw2b)
    acc2 = acc2.reshape(2, 2, NP, OUT_D // 2)
    den2 = den2.reshape(2, 2, NP)[:, 0, :].reshape(2, NP, 1)

    return _tc3(acc2, den2, b2.reshape(1, OUT_D))


# speedup vs baseline: 27.1788x; 1.3692x over previous
import jax
import jax.numpy as jnp
from jax import lax
from jax.experimental import pallas as pl
from jax.experimental.pallas import tpu as pltpu
from jax.experimental.pallas import tpu_sc as plsc

N = 10000
E = 320000
IN_D = 128
HID_D = 64
NHEAD = 8
OUT_D = 128

NC = 2
NS = 16
NW = NC * NS
EPT = E // NW
BLK = 80
NBLK = EPT // BLK
CHK = 25
NCHK = NBLK // CHK
NP = 10240
NROW = NP // NS
ZR = 64

_f32 = jnp.float32


def _tc1_body(x_ref, w1_ref, asr_ref, adr_ref, *refs):
    hw_refs = refs[0:NHEAD]
    as_refs = refs[NHEAD:2 * NHEAD]
    ad_refs = refs[2 * NHEAD:3 * NHEAD]
    cc_ref = refs[3 * NHEAD]
    mxs_ref, mxd_ref = refs[3 * NHEAD + 1], refs[3 * NHEAD + 2]
    i = pl.program_id(0)
    nb = pl.num_programs(0)
    hb = jnp.dot(x_ref[...], w1_ref[...], preferred_element_type=_f32)
    for h in range(NHEAD):
        hh = hb[:, h * HID_D:(h + 1) * HID_D]
        hw_refs[h][...] = hh
        sv = jnp.sum(hh * asr_ref[h:h + 1, :], axis=1, keepdims=True)
        dv = jnp.sum(hh * adr_ref[h:h + 1, :], axis=1, keepdims=True)
        as_refs[h][...] = sv
        ad_refs[h][...] = dv
        ms = jnp.max(sv)
        md = jnp.max(dv)

        @pl.when(i == 0)
        def _(h=h, ms=ms, md=md):
            mxs_ref[h:h + 1, :] = jnp.full((1, 128), ms, _f32)
            mxd_ref[h:h + 1, :] = jnp.full((1, 128), md, _f32)

        @pl.when(i > 0)
        def _(h=h, ms=ms, md=md):
            mxs_ref[h:h + 1, :] = jnp.maximum(mxs_ref[h:h + 1, :], ms)
            mxd_ref[h:h + 1, :] = jnp.maximum(mxd_ref[h:h + 1, :], md)

    @pl.when(i == nb - 1)
    def _():
        t = mxs_ref[...] + mxd_ref[...]
        cc_ref[...] = jnp.maximum(t, 0.2 * t)


def _tc1(x, W1, a_src1, a_dst1):
    bn = 1000
    grid = (N // bn,)
    outs = (
        [jax.ShapeDtypeStruct((N, HID_D), _f32)] * NHEAD
        + [jax.ShapeDtypeStruct((N, 1), _f32)] * (2 * NHEAD)
        + [jax.ShapeDtypeStruct((NHEAD, 128), _f32)]
    )
    out_specs = (
        [pl.BlockSpec((bn, HID_D), lambda i: (i, 0))] * NHEAD
        + [pl.BlockSpec((bn, 1), lambda i: (i, 0))] * (2 * NHEAD)
        + [pl.BlockSpec((NHEAD, 128), lambda i: (0, 0))]
    )
    return pl.pallas_call(
        _tc1_body,
        grid=grid,
        in_specs=[
            pl.BlockSpec((bn, IN_D), lambda i: (i, 0)),
            pl.BlockSpec((IN_D, NHEAD * HID_D), lambda i: (0, 0)),
            pl.BlockSpec((NHEAD, HID_D), lambda i: (0, 0)),
            pl.BlockSpec((NHEAD, HID_D), lambda i: (0, 0)),
        ],
        out_specs=out_specs,
        out_shape=outs,
        scratch_shapes=[
            pltpu.VMEM((NHEAD, 128), _f32),
            pltpu.VMEM((NHEAD, 128), _f32),
        ],
    )(x, W1, a_src1, a_dst1)


NDR = NP // 16
DRW = NROW // 16


def _make_sc_agg(H, D):
    GD = D // 16
    mesh = plsc.VectorSubcoreMesh(core_axis_name="c", subcore_axis_name="s")

    def body(src_h, dst_h, cc_h, *rest):
        as_hs = rest[0:H]
        ad_hs = rest[H:2 * H]
        hw_hs = rest[2 * H:3 * H]
        acc_o = rest[3 * H]
        den_o = rest[3 * H + 1]
        (srcb, dstb, as_v, ad_v, cc_v, rows, exb, denb, rows_den, zbuf,
         zden, acc_sh, den_sh, sem_g, sem_s, sem_d) = rest[3 * H + 2:]
        c = lax.axis_index("c")
        s = lax.axis_index("s")
        wid = s * NC + c
        pltpu.sync_copy(cc_h, cc_v)
        lanes = lax.iota(jnp.int32, 16)

        def zrow(r, carry):
            for k in range(GD):
                zbuf[r, pl.ds(k * 16, 16)] = jnp.zeros((16,), _f32)
            return carry

        lax.fori_loop(0, ZR, zrow, 0)

        def zdrow(r, carry):
            zden[r, pl.ds(0, 16)] = jnp.zeros((16,), _f32)
            return carry

        lax.fori_loop(0, DRW, zdrow, 0)

        for h in range(H):
            pltpu.sync_copy(as_hs[h], as_v)
            pltpu.sync_copy(ad_hs[h], ad_v)
            ccx = cc_v[pl.ds(h * 16, 16)]
            for z in range(NROW // ZR):
                pltpu.sync_copy(zbuf, acc_sh.at[pl.ds(s * NROW + z * ZR, ZR)])
            pltpu.sync_copy(zden, den_sh.at[pl.ds(s * DRW, DRW)])
            plsc.subcore_barrier()
            hw = hw_hs[h]

            def blk(b, carry, ccx=ccx, hw=hw):
                p = jnp.bitwise_and(b, 1)
                q = 1 - p

                @pl.when(b >= 2)
                def _():
                    pltpu.make_async_copy(
                        rows_den.at[p], den_sh.at[denb.at[p]],
                        sem_d.at[p]).wait()

                for g in range(BLK // 16):
                    sidx = srcb[b, pl.ds(g * 16, 16)]
                    didx = dstb[b, pl.ds(g * 16, 16)]
                    sv = plsc.load_gather(as_v, [sidx])
                    dv = plsc.load_gather(ad_v, [didx])
                    t = sv + dv
                    e = jnp.maximum(t, 0.2 * t)
                    exb[pl.ds(g * 16, 16)] = jnp.exp(e - ccx)
                    denb[p, pl.ds(g * 16, 16)] = (
                        lax.shift_right_logical(didx, 4))
                pltpu.make_async_copy(hw.at[srcb.at[b]], rows.at[p],
                                      sem_g.at[p]).wait()

                @pl.when(b >= 1)
                def _():
                    pltpu.make_async_copy(
                        rows.at[q], acc_sh.at[dstb.at[b - 1]],
                        sem_s.at[q]).wait()

                @pl.when(b < CHK - 1)
                def _():
                    pltpu.async_copy(hw.at[srcb.at[b + 1]], rows.at[q],
                                     sem_g.at[q])

                rdp = rows_den.at[p]

                def zr2(j, inner):
                    for u in range(4):
                        rdp[j * 4 + u, pl.ds(0, 16)] = jnp.zeros((16,), _f32)
                    return inner

                lax.fori_loop(0, BLK // 4, zr2, 0)
                for g in range(BLK // 16):
                    didx = dstb[b, pl.ds(g * 16, 16)]
                    ex = exb[pl.ds(g * 16, 16)]
                    plsc.store_scatter(
                        rdp, [g * 16 + lanes, jnp.bitwise_and(didx, 15)], ex)
                rp = rows.at[p]

                def srow(j, inner):
                    for u in range(4):
                        i = j * 4 + u
                        bc = plsc.load_gather(
                            exb, [jnp.full((16,), i, jnp.int32)])
                        for k in range(GD):
                            rp[i, pl.ds(k * 16, 16)] = (
                                rp[i, pl.ds(k * 16, 16)] * bc)
                    return inner

                lax.fori_loop(0, BLK // 4, srow, 0)
                pltpu.async_copy(rp, acc_sh.at[dstb.at[b]], sem_s.at[p],
                                 add=True)
                pltpu.async_copy(rdp, den_sh.at[denb.at[p]], sem_d.at[p],
                                 add=True)
                return carry

            def chunk(kc, carry, hw=hw, blk=blk):
                pltpu.sync_copy(src_h.at[wid, pl.ds(kc * CHK, CHK)], srcb)
                pltpu.sync_copy(dst_h.at[wid, pl.ds(kc * CHK, CHK)], dstb)
                pltpu.async_copy(hw.at[srcb.at[0]], rows.at[0], sem_g.at[0])
                lax.fori_loop(0, CHK, blk, 0)
                pltpu.make_async_copy(
                    rows.at[(CHK - 1) & 1], acc_sh.at[dstb.at[CHK - 1]],
                    sem_s.at[(CHK - 1) & 1]).wait()
                for par in ((CHK - 1) & 1, (CHK - 2) & 1):
                    pltpu.make_async_copy(
                        rows_den.at[par], den_sh.at[denb.at[par]],
                        sem_d.at[par]).wait()
                return carry

            lax.fori_loop(0, NCHK, chunk, 0)
            plsc.subcore_barrier()
            off = (c * H + h) * NP + s * NROW
            pltpu.sync_copy(acc_sh.at[pl.ds(s * NROW, NROW)],
                            acc_o.at[pl.ds(off, NROW)])
            doff = (c * H + h) * NDR + s * DRW
            pltpu.sync_copy(den_sh.at[pl.ds(s * DRW, DRW)],
                            den_o.at[pl.ds(doff, DRW)])

    scratch = [
        pltpu.VMEM((CHK, BLK), jnp.int32),
        pltpu.VMEM((CHK, BLK), jnp.int32),
        pltpu.VMEM((NP,), _f32),
        pltpu.VMEM((NP,), _f32),
        pltpu.VMEM((H * 16,), _f32),
        pltpu.VMEM((2, BLK, D), _f32),
        pltpu.VMEM((128,), _f32),
        pltpu.VMEM((2, BLK), jnp.int32),
        pltpu.VMEM((2, BLK, 16), _f32),
        pltpu.VMEM((ZR, D), _f32),
        pltpu.VMEM((DRW, 16), _f32),
        pltpu.VMEM_SHARED((NP, D), _f32),
        pltpu.VMEM_SHARED((NDR, 16), _f32),
        pltpu.SemaphoreType.DMA((2,)),
        pltpu.SemaphoreType.DMA((2,)),
        pltpu.SemaphoreType.DMA((2,)),
    ]
    return pl.kernel(
        body,
        out_type=(jax.ShapeDtypeStruct((2 * H * NP, D), _f32),
                  jax.ShapeDtypeStruct((2 * H * NDR, 16), _f32)),
        mesh=mesh,
        scratch_types=scratch,
        compiler_params=pltpu.CompilerParams(
            needs_layout_passes=False, use_tc_tiling_on_sc=False),
    )


def _tc2_body(acc_ref, den_ref, b1_ref, w2_ref, asw_ref, adw_ref,
              hw2a_ref, hw2b_ref, as2_ref, ad2_ref, cc2_ref,
              mxs_ref, mxd_ref):
    i = pl.program_id(0)
    nb = pl.num_programs(0)
    cols = []
    for h in range(NHEAD):
        num = acc_ref[0, h, :, :] + acc_ref[1, h, :, :]
        den = den_ref[0, h, :, :] + den_ref[1, h, :, :]
        o = num / (den + 1e-16) + b1_ref[0:1, h * HID_D:(h + 1) * HID_D]
        o = jnp.where(o > 0, o, jnp.exp(o) - 1.0)
        cols.append(o)
    h1 = jnp.concatenate(cols, axis=1)
    hw2 = jnp.dot(h1, w2_ref[...], preferred_element_type=_f32)
    hw2a_ref[...] = hw2[:, 0:OUT_D // 2]
    hw2b_ref[...] = hw2[:, OUT_D // 2:OUT_D]
    sv = jnp.sum(hw2 * asw_ref[...], axis=1, keepdims=True)
    dv = jnp.sum(hw2 * adw_ref[...], axis=1, keepdims=True)
    as2_ref[...] = sv
    ad2_ref[...] = dv
    ms = jnp.max(sv)
    md = jnp.max(dv)

    @pl.when(i == 0)
    def _():
        mxs_ref[...] = jnp.full((8, 128), ms, _f32)
        mxd_ref[...] = jnp.full((8, 128), md, _f32)

    @pl.when(i > 0)
    def _():
        mxs_ref[...] = jnp.maximum(mxs_ref[...], ms)
        mxd_ref[...] = jnp.maximum(mxd_ref[...], md)

    @pl.when(i == nb - 1)
    def _():
        t = mxs_ref[...] + mxd_ref[...]
        cc2_ref[...] = jnp.maximum(t, 0.2 * t)


def _tc2(acc1, den1, b1, W2, a_src2, a_dst2):
    bn = 1000
    grid = (N // bn,)
    return pl.pallas_call(
        _tc2_body,
        grid=grid,
        in_specs=[
            pl.BlockSpec((2, NHEAD, bn, HID_D), lambda i: (0, 0, i, 0)),
            pl.BlockSpec((2, NHEAD, bn, 1), lambda i: (0, 0, i, 0)),
            pl.BlockSpec((1, NHEAD * HID_D), lambda i: (0, 0)),
            pl.BlockSpec((NHEAD * HID_D, OUT_D), lambda i: (0, 0)),
            pl.BlockSpec((1, OUT_D), lambda i: (0, 0)),
            pl.BlockSpec((1, OUT_D), lambda i: (0, 0)),
        ],
        out_specs=[
            pl.BlockSpec((bn, OUT_D // 2), lambda i: (i, 0)),
            pl.BlockSpec((bn, OUT_D // 2), lambda i: (i, 0)),
            pl.BlockSpec((bn, 1), lambda i: (i, 0)),
            pl.BlockSpec((bn, 1), lambda i: (i, 0)),
            pl.BlockSpec((8, 128), lambda i: (0, 0)),
        ],
        out_shape=[
            jax.ShapeDtypeStruct((N, OUT_D // 2), _f32),
            jax.ShapeDtypeStruct((N, OUT_D // 2), _f32),
            jax.ShapeDtypeStruct((N, 1), _f32),
            jax.ShapeDtypeStruct((N, 1), _f32),
            jax.ShapeDtypeStruct((8, 128), _f32),
        ],
        scratch_shapes=[
            pltpu.VMEM((8, 128), _f32),
            pltpu.VMEM((8, 128), _f32),
        ],
    )(acc1, den1, b1, W2, a_src2, a_dst2)


def _tc3_body(acc_ref, den_ref, b2_ref, o_ref):
    num = jnp.concatenate(
        [acc_ref[0, 0, :, :] + acc_ref[1, 0, :, :],
         acc_ref[0, 1, :, :] + acc_ref[1, 1, :, :]], axis=1)
    den = den_ref[0, :, :] + den_ref[1, :, :]
    o = num / (den + 1e-16) + b2_ref[0:1, :]
    m = jnp.max(o, axis=1, keepdims=True)
    sh = o - m
    lse = jnp.log(jnp.sum(jnp.exp(sh), axis=1, keepdims=True))
    o_ref[...] = sh - lse


def _tc3(acc2, den2, b2):
    bn = 1000
    grid = (N // bn,)
    return pl.pallas_call(
        _tc3_body,
        grid=grid,
        in_specs=[
            pl.BlockSpec((2, 2, bn, OUT_D // 2), lambda i: (0, 0, i, 0)),
            pl.BlockSpec((2, bn, 1), lambda i: (0, i, 0)),
            pl.BlockSpec((1, OUT_D), lambda i: (0, 0)),
        ],
        out_specs=pl.BlockSpec((bn, OUT_D), lambda i: (i, 0)),
        out_shape=jax.ShapeDtypeStruct((N, OUT_D), _f32),
    )(acc2, den2, b2)


def kernel(x, edge_index, W1, a_src1, a_dst1, b1, W2, a_src2, a_dst2, b2):
    src = edge_index[0].astype(jnp.int32).reshape(NW, NBLK, BLK)
    dst = edge_index[1].astype(jnp.int32).reshape(NW, NBLK, BLK)

    t1 = _tc1(x, W1, a_src1, a_dst1)
    hw1 = t1[0:NHEAD]
    as1 = [jnp.pad(a.reshape(N), (0, NP - N)) for a in t1[NHEAD:2 * NHEAD]]
    ad1 = [jnp.pad(a.reshape(N), (0, NP - N)) for a in t1[2 * NHEAD:3 * NHEAD]]
    cc1x = t1[3 * NHEAD][:, 0:16].reshape(NHEAD * 16)

    agg1 = _make_sc_agg(NHEAD, HID_D)
    acc1, den1 = agg1(src, dst, cc1x, *as1, *ad1, *hw1)
    acc1 = acc1.reshape(2, NHEAD, NP, HID_D)
    den1 = den1.reshape(2, NHEAD, NP, 1)

    hw2a, hw2b, as2, ad2, cc2f = _tc2(acc1, den1,
                                      b1.reshape(1, NHEAD * HID_D), W2,
                                      a_src2, a_dst2)
    cc2 = cc2f[0, 0:16]
    cc2x = jnp.concatenate([cc2, cc2])

    as2p = jnp.pad(as2.reshape(N), (0, NP - N))
    ad2p = jnp.pad(ad2.reshape(N), (0, NP - N))
    agg2 = _make_sc_agg(2, OUT_D // 2)
    acc2, den2 = agg2(src, dst, cc2x, as2p, as2p, ad2p, ad2p, hw2a, hw2b)
    acc2 = acc2.reshape(2, 2, NP, OUT_D // 2)
    den2 = den2.reshape(2, 2, NP)[:, 0, :].reshape(2, NP, 1)

    return _tc3(acc2, den2, b2.reshape(1, OUT_D))


# speedup vs baseline: 27.2994x; 1.0044x over previous
import jax
import jax.numpy as jnp
from jax import lax
from jax.experimental import pallas as pl
from jax.experimental.pallas import tpu as pltpu
from jax.experimental.pallas import tpu_sc as plsc

N = 10000
E = 320000
IN_D = 128
HID_D = 64
NHEAD = 8
OUT_D = 128

NC = 2
NS = 16
NW = NC * NS
EPT = E // NW
BLK = 80
NBLK = EPT // BLK
CHK = 25
NCHK = NBLK // CHK
NP = 10240
NROW = NP // NS
ZR = 64

_f32 = jnp.float32


def _tc1_body(x_ref, w1_ref, asr_ref, adr_ref, *refs):
    hw_refs = refs[0:NHEAD]
    as_refs = refs[NHEAD:2 * NHEAD]
    ad_refs = refs[2 * NHEAD:3 * NHEAD]
    cc_ref = refs[3 * NHEAD]
    mxs_ref, mxd_ref = refs[3 * NHEAD + 1], refs[3 * NHEAD + 2]
    i = pl.program_id(0)
    nb = pl.num_programs(0)
    hb = jnp.dot(x_ref[...], w1_ref[...], preferred_element_type=_f32)
    for h in range(NHEAD):
        hh = hb[:, h * HID_D:(h + 1) * HID_D]
        hw_refs[h][...] = hh
        sv = jnp.sum(hh * asr_ref[h:h + 1, :], axis=1, keepdims=True)
        dv = jnp.sum(hh * adr_ref[h:h + 1, :], axis=1, keepdims=True)
        as_refs[h][...] = sv
        ad_refs[h][...] = dv
        ms = jnp.max(sv)
        md = jnp.max(dv)

        @pl.when(i == 0)
        def _(h=h, ms=ms, md=md):
            mxs_ref[h:h + 1, :] = jnp.full((1, 128), ms, _f32)
            mxd_ref[h:h + 1, :] = jnp.full((1, 128), md, _f32)

        @pl.when(i > 0)
        def _(h=h, ms=ms, md=md):
            mxs_ref[h:h + 1, :] = jnp.maximum(mxs_ref[h:h + 1, :], ms)
            mxd_ref[h:h + 1, :] = jnp.maximum(mxd_ref[h:h + 1, :], md)

    @pl.when(i == nb - 1)
    def _():
        t = mxs_ref[...] + mxd_ref[...]
        cc_ref[...] = jnp.maximum(t, 0.2 * t)


def _tc1(x, W1, a_src1, a_dst1):
    bn = 1000
    grid = (N // bn,)
    outs = (
        [jax.ShapeDtypeStruct((N, HID_D), _f32)] * NHEAD
        + [jax.ShapeDtypeStruct((NP, 1), _f32)] * (2 * NHEAD)
        + [jax.ShapeDtypeStruct((NHEAD, 128), _f32)]
    )
    out_specs = (
        [pl.BlockSpec((bn, HID_D), lambda i: (i, 0))] * NHEAD
        + [pl.BlockSpec((bn, 1), lambda i: (i, 0))] * (2 * NHEAD)
        + [pl.BlockSpec((NHEAD, 128), lambda i: (0, 0))]
    )
    return pl.pallas_call(
        _tc1_body,
        grid=grid,
        in_specs=[
            pl.BlockSpec((bn, IN_D), lambda i: (i, 0)),
            pl.BlockSpec((IN_D, NHEAD * HID_D), lambda i: (0, 0)),
            pl.BlockSpec((NHEAD, HID_D), lambda i: (0, 0)),
            pl.BlockSpec((NHEAD, HID_D), lambda i: (0, 0)),
        ],
        out_specs=out_specs,
        out_shape=outs,
        scratch_shapes=[
            pltpu.VMEM((NHEAD, 128), _f32),
            pltpu.VMEM((NHEAD, 128), _f32),
        ],
    )(x, W1, a_src1, a_dst1)


NDR = NP // 16
DRW = NROW // 16


def _make_sc_agg(H, D):
    GD = D // 16
    mesh = plsc.VectorSubcoreMesh(core_axis_name="c", subcore_axis_name="s")

    def body(src_h, dst_h, cc_h, *rest):
        as_hs = rest[0:H]
        ad_hs = rest[H:2 * H]
        hw_hs = rest[2 * H:3 * H]
        acc_o = rest[3 * H]
        den_o = rest[3 * H + 1]
        (srcb, dstb, as_v, ad_v, cc_v, rows, exb, denb, rows_den, zbuf,
         zden, acc_sh, den_sh, sem_g, sem_s, sem_d) = rest[3 * H + 2:]
        c = lax.axis_index("c")
        s = lax.axis_index("s")
        wid = s * NC + c
        pltpu.sync_copy(cc_h, cc_v)
        lanes = lax.iota(jnp.int32, 16)

        def zrow(r, carry):
            for k in range(GD):
                zbuf[r, pl.ds(k * 16, 16)] = jnp.zeros((16,), _f32)
            return carry

        lax.fori_loop(0, ZR, zrow, 0)

        def zdrow(r, carry):
            zden[r, pl.ds(0, 16)] = jnp.zeros((16,), _f32)
            return carry

        lax.fori_loop(0, DRW, zdrow, 0)

        for h in range(H):
            pltpu.sync_copy(as_hs[h], as_v)
            pltpu.sync_copy(ad_hs[h], ad_v)
            ccx = cc_v[pl.ds(h * 16, 16)]
            for z in range(NROW // ZR):
                pltpu.sync_copy(zbuf, acc_sh.at[pl.ds(s * NROW + z * ZR, ZR)])
            pltpu.sync_copy(zden, den_sh.at[pl.ds(s * DRW, DRW)])
            plsc.subcore_barrier()
            hw = hw_hs[h]

            def blk(b, carry, ccx=ccx, hw=hw):
                p = jnp.bitwise_and(b, 1)
                q = 1 - p

                @pl.when(b >= 2)
                def _():
                    pltpu.make_async_copy(
                        rows_den.at[p], den_sh.at[denb.at[p]],
                        sem_d.at[p]).wait()

                for g in range(BLK // 16):
                    sidx = srcb[b, pl.ds(g * 16, 16)]
                    didx = dstb[b, pl.ds(g * 16, 16)]
                    sv = plsc.load_gather(as_v, [sidx])
                    dv = plsc.load_gather(ad_v, [didx])
                    t = sv + dv
                    e = jnp.maximum(t, 0.2 * t)
                    exb[pl.ds(g * 16, 16)] = jnp.exp(e - ccx)
                    denb[p, pl.ds(g * 16, 16)] = (
                        lax.shift_right_logical(didx, 4))
                pltpu.make_async_copy(hw.at[srcb.at[b]], rows.at[p],
                                      sem_g.at[p]).wait()

                @pl.when(b >= 1)
                def _():
                    pltpu.make_async_copy(
                        rows.at[q], acc_sh.at[dstb.at[b - 1]],
                        sem_s.at[q]).wait()

                @pl.when(b < CHK - 1)
                def _():
                    pltpu.async_copy(hw.at[srcb.at[b + 1]], rows.at[q],
                                     sem_g.at[q])

                rdp = rows_den.at[p]

                def zr2(j, inner):
                    for u in range(4):
                        rdp[j * 4 + u, pl.ds(0, 16)] = jnp.zeros((16,), _f32)
                    return inner

                lax.fori_loop(0, BLK // 4, zr2, 0)
                for g in range(BLK // 16):
                    didx = dstb[b, pl.ds(g * 16, 16)]
                    ex = exb[pl.ds(g * 16, 16)]
                    plsc.store_scatter(
                        rdp, [g * 16 + lanes, jnp.bitwise_and(didx, 15)], ex)
                rp = rows.at[p]

                def srow(j, inner):
                    for u in range(8):
                        i = j * 8 + u
                        bc = plsc.load_gather(
                            exb, [jnp.full((16,), i, jnp.int32)])
                        for k in range(GD):
                            rp[i, pl.ds(k * 16, 16)] = (
                                rp[i, pl.ds(k * 16, 16)] * bc)
                    return inner

                lax.fori_loop(0, BLK // 8, srow, 0)
                pltpu.async_copy(rp, acc_sh.at[dstb.at[b]], sem_s.at[p],
                                 add=True)
                pltpu.async_copy(rdp, den_sh.at[denb.at[p]], sem_d.at[p],
                                 add=True)
                return carry

            def chunk(kc, carry, hw=hw, blk=blk):
                pltpu.sync_copy(src_h.at[wid, pl.ds(kc * CHK, CHK)], srcb)
                pltpu.sync_copy(dst_h.at[wid, pl.ds(kc * CHK, CHK)], dstb)
                pltpu.async_copy(hw.at[srcb.at[0]], rows.at[0], sem_g.at[0])
                lax.fori_loop(0, CHK, blk, 0)
                pltpu.make_async_copy(
                    rows.at[(CHK - 1) & 1], acc_sh.at[dstb.at[CHK - 1]],
                    sem_s.at[(CHK - 1) & 1]).wait()
                for par in ((CHK - 1) & 1, (CHK - 2) & 1):
                    pltpu.make_async_copy(
                        rows_den.at[par], den_sh.at[denb.at[par]],
                        sem_d.at[par]).wait()
                return carry

            lax.fori_loop(0, NCHK, chunk, 0)
            plsc.subcore_barrier()
            off = (c * H + h) * NP + s * NROW
            pltpu.sync_copy(acc_sh.at[pl.ds(s * NROW, NROW)],
                            acc_o.at[pl.ds(off, NROW)])
            doff = (c * H + h) * NDR + s * DRW
            pltpu.sync_copy(den_sh.at[pl.ds(s * DRW, DRW)],
                            den_o.at[pl.ds(doff, DRW)])

    scratch = [
        pltpu.VMEM((CHK, BLK), jnp.int32),
        pltpu.VMEM((CHK, BLK), jnp.int32),
        pltpu.VMEM((NP,), _f32),
        pltpu.VMEM((NP,), _f32),
        pltpu.VMEM((H * 16,), _f32),
        pltpu.VMEM((2, BLK, D), _f32),
        pltpu.VMEM((128,), _f32),
        pltpu.VMEM((2, BLK), jnp.int32),
        pltpu.VMEM((2, BLK, 16), _f32),
        pltpu.VMEM((ZR, D), _f32),
        pltpu.VMEM((DRW, 16), _f32),
        pltpu.VMEM_SHARED((NP, D), _f32),
        pltpu.VMEM_SHARED((NDR, 16), _f32),
        pltpu.SemaphoreType.DMA((2,)),
        pltpu.SemaphoreType.DMA((2,)),
        pltpu.SemaphoreType.DMA((2,)),
    ]
    return pl.kernel(
        body,
        out_type=(jax.ShapeDtypeStruct((2 * H * NP, D), _f32),
                  jax.ShapeDtypeStruct((2 * H * NDR, 16), _f32)),
        mesh=mesh,
        scratch_types=scratch,
        compiler_params=pltpu.CompilerParams(
            needs_layout_passes=False, use_tc_tiling_on_sc=False),
    )


def _tc2_body(acc_ref, den_ref, b1_ref, w2_ref, asw_ref, adw_ref,
              hw2a_ref, hw2b_ref, as2_ref, ad2_ref, cc2_ref,
              mxs_ref, mxd_ref):
    i = pl.program_id(0)
    nb = pl.num_programs(0)
    cols = []
    for h in range(NHEAD):
        num = acc_ref[0, h, :, :] + acc_ref[1, h, :, :]
        den = den_ref[0, h, :, :] + den_ref[1, h, :, :]
        o = num / (den + 1e-16) + b1_ref[0:1, h * HID_D:(h + 1) * HID_D]
        o = jnp.where(o > 0, o, jnp.exp(o) - 1.0)
        cols.append(o)
    h1 = jnp.concatenate(cols, axis=1)
    hw2 = jnp.dot(h1, w2_ref[...], preferred_element_type=_f32)
    hw2a_ref[...] = hw2[:, 0:OUT_D // 2]
    hw2b_ref[...] = hw2[:, OUT_D // 2:OUT_D]
    sv = jnp.sum(hw2 * asw_ref[...], axis=1, keepdims=True)
    dv = jnp.sum(hw2 * adw_ref[...], axis=1, keepdims=True)
    as2_ref[...] = sv
    ad2_ref[...] = dv
    ms = jnp.max(sv)
    md = jnp.max(dv)

    @pl.when(i == 0)
    def _():
        mxs_ref[...] = jnp.full((8, 128), ms, _f32)
        mxd_ref[...] = jnp.full((8, 128), md, _f32)

    @pl.when(i > 0)
    def _():
        mxs_ref[...] = jnp.maximum(mxs_ref[...], ms)
        mxd_ref[...] = jnp.maximum(mxd_ref[...], md)

    @pl.when(i == nb - 1)
    def _():
        t = mxs_ref[...] + mxd_ref[...]
        cc2_ref[...] = jnp.maximum(t, 0.2 * t)


def _tc2(acc1, den1, b1, W2, a_src2, a_dst2):
    bn = 1000
    grid = (N // bn,)
    return pl.pallas_call(
        _tc2_body,
        grid=grid,
        in_specs=[
            pl.BlockSpec((2, NHEAD, bn, HID_D), lambda i: (0, 0, i, 0)),
            pl.BlockSpec((2, NHEAD, bn, 1), lambda i: (0, 0, i, 0)),
            pl.BlockSpec((1, NHEAD * HID_D), lambda i: (0, 0)),
            pl.BlockSpec((NHEAD * HID_D, OUT_D), lambda i: (0, 0)),
            pl.BlockSpec((1, OUT_D), lambda i: (0, 0)),
            pl.BlockSpec((1, OUT_D), lambda i: (0, 0)),
        ],
        out_specs=[
            pl.BlockSpec((bn, OUT_D // 2), lambda i: (i, 0)),
            pl.BlockSpec((bn, OUT_D // 2), lambda i: (i, 0)),
            pl.BlockSpec((bn, 1), lambda i: (i, 0)),
            pl.BlockSpec((bn, 1), lambda i: (i, 0)),
            pl.BlockSpec((8, 128), lambda i: (0, 0)),
        ],
        out_shape=[
            jax.ShapeDtypeStruct((N, OUT_D // 2), _f32),
            jax.ShapeDtypeStruct((N, OUT_D // 2), _f32),
            jax.ShapeDtypeStruct((NP, 1), _f32),
            jax.ShapeDtypeStruct((NP, 1), _f32),
            jax.ShapeDtypeStruct((8, 128), _f32),
        ],
        scratch_shapes=[
            pltpu.VMEM((8, 128), _f32),
            pltpu.VMEM((8, 128), _f32),
        ],
    )(acc1, den1, b1, W2, a_src2, a_dst2)


def _tc3_body(acc_ref, den_ref, b2_ref, o_ref):
    num = jnp.concatenate(
        [acc_ref[0, 0, :, :] + acc_ref[1, 0, :, :],
         acc_ref[0, 1, :, :] + acc_ref[1, 1, :, :]], axis=1)
    den = den_ref[0, :, :] + den_ref[1, :, :]
    o = num / (den + 1e-16) + b2_ref[0:1, :]
    m = jnp.max(o, axis=1, keepdims=True)
    sh = o - m
    lse = jnp.log(jnp.sum(jnp.exp(sh), axis=1, keepdims=True))
    o_ref[...] = sh - lse


def _tc3(acc2, den2, b2):
    bn = 1000
    grid = (N // bn,)
    return pl.pallas_call(
        _tc3_body,
        grid=grid,
        in_specs=[
            pl.BlockSpec((2, 2, bn, OUT_D // 2), lambda i: (0, 0, i, 0)),
            pl.BlockSpec((2, bn, 1), lambda i: (0, i, 0)),
            pl.BlockSpec((1, OUT_D), lambda i: (0, 0)),
        ],
        out_specs=pl.BlockSpec((bn, OUT_D), lambda i: (i, 0)),
        out_shape=jax.ShapeDtypeStruct((N, OUT_D), _f32),
    )(acc2, den2, b2)


def kernel(x, edge_index, W1, a_src1, a_dst1, b1, W2, a_src2, a_dst2, b2):
    src = edge_index[0].astype(jnp.int32).reshape(NW, NBLK, BLK)
    dst = edge_index[1].astype(jnp.int32).reshape(NW, NBLK, BLK)

    t1 = _tc1(x, W1, a_src1, a_dst1)
    hw1 = t1[0:NHEAD]
    as1 = [a.reshape(NP) for a in t1[NHEAD:2 * NHEAD]]
    ad1 = [a.reshape(NP) for a in t1[2 * NHEAD:3 * NHEAD]]
    cc1x = t1[3 * NHEAD][:, 0:16].reshape(NHEAD * 16)

    agg1 = _make_sc_agg(NHEAD, HID_D)
    acc1, den1 = agg1(src, dst, cc1x, *as1, *ad1, *hw1)
    acc1 = acc1.reshape(2, NHEAD, NP, HID_D)
    den1 = den1.reshape(2, NHEAD, NP, 1)

    hw2a, hw2b, as2, ad2, cc2f = _tc2(acc1, den1,
                                      b1.reshape(1, NHEAD * HID_D), W2,
                                      a_src2, a_dst2)
    cc2 = cc2f[0, 0:16]
    cc2x = jnp.concatenate([cc2, cc2])

    as2p = as2.reshape(NP)
    ad2p = ad2.reshape(NP)
    agg2 = _make_sc_agg(2, OUT_D // 2)
    acc2, den2 = agg2(src, dst, cc2x, as2p, as2p, ad2p, ad2p, hw2a, hw2b)
    acc2 = acc2.reshape(2, 2, NP, OUT_D // 2)
    den2 = den2.reshape(2, 2, NP)[:, 0, :].reshape(2, NP, 1)

    return _tc3(acc2, den2, b2.reshape(1, OUT_D))
